# baseline (speedup 1.0000x reference)
import functools

import jax
import jax.numpy as jnp
from jax import lax
from jax.experimental import pallas as pl
from jax.experimental.pallas import tpu as pltpu
from jax.experimental.pallas import tpu_sc as plsc

VOCAB = 1000000
DIM = 32
B = 16384
L = 50

NUM_CORES = 2
NUM_SUBCORES = 16
NUM_WORKERS = NUM_CORES * NUM_SUBCORES

GDIM = 128

ROWS_PER_WORKER = B // NUM_WORKERS
CHUNK_ROWS = 16
CHUNKS = ROWS_PER_WORKER // CHUNK_ROWS
IDX_PER_CHUNK = CHUNK_ROWS * L
GATHER_BATCH = 80
GATHERS = IDX_PER_CHUNK // GATHER_BATCH


_GATHER_DNUMS = lax.GatherDimensionNumbers(
    offset_dims=(), collapsed_slice_dims=(0,), start_index_map=(0,))


def _perm16(v, perm):
    return lax.gather(v, perm[:, None], _GATHER_DNUMS, (1,),
                      mode=lax.GatherScatterMode.PROMISE_IN_BOUNDS)


def _lane_sum(v, lanes):
    for k in (1, 2, 4, 8):
        v = v + _perm16(v, lanes ^ k)
    return v


def _tower_kernel(xflat, table, out, idxg_v, idxf_v, rows_v, outc_v,
                  t0_v, sem):
    wid = lax.axis_index("s") * NUM_CORES + lax.axis_index("c")

    pltpu.sync_copy(table.at[0], t0_v)

    def chunk_body(c, carry):
        crow = wid * ROWS_PER_WORKER + c * CHUNK_ROWS
        foff = crow * L

        icps = []
        for j in range(GATHERS):
            icps.append(pltpu.async_copy(
                xflat.at[pl.ds(pl.multiple_of(foff + j * GATHER_BATCH, 8),
                               GATHER_BATCH)],
                idxg_v.at[j], sem))
        for cp in icps:
            cp.wait()
        pltpu.sync_copy(xflat.at[pl.ds(pl.multiple_of(foff, 8),
                                       IDX_PER_CHUNK)], idxf_v)

        cps = []
        for j in range(GATHERS):
            cps.append(pltpu.async_copy(
                table.at[idxg_v.at[j]],
                rows_v.at[pl.ds(j * GATHER_BATCH, GATHER_BATCH)],
                sem))
        for cp in cps:
            cp.wait()

        lanes = lax.iota(jnp.int32, 16)
        one = jnp.full((16,), 1.0, jnp.float32)
        zrow = jnp.full((16,), 0.0, jnp.float32)
        lt2 = jnp.where(lanes < 2, one, zrow)
        t00 = t0_v[pl.ds(0, 16)]
        t01 = t0_v[pl.ds(16, 16)]

        def row_body(r, rcarry):
            fo = r * L
            zero = jnp.zeros((16,), jnp.float32)
            a0 = [zero, zero, zero, zero]
            a1 = [zero, zero, zero, zero]
            for l in range(L):
                a0[l & 3] = a0[l & 3] + rows_v[fo + l, pl.ds(0, 16)]
                a1[l & 3] = a1[l & 3] + rows_v[fo + l, pl.ds(16, 16)]
            acc0 = (a0[0] + a0[1]) + (a0[2] + a0[3])
            acc1 = (a1[0] + a1[1]) + (a1[2] + a1[3])

            i0 = idxf_v[pl.ds(fo, 16)]
            i1 = idxf_v[pl.ds(fo + 16, 16)]
            i2 = idxf_v[pl.ds(fo + 32, 16)]
            i3 = idxf_v[pl.ds(fo + 34, 16)]
            m0 = jnp.where(i0 != 0, one, zrow)
            m1 = jnp.where(i1 != 0, one, zrow)
            m2 = jnp.where(i2 != 0, lt2, zrow)
            m3 = jnp.where(i3 != 0, one, zrow)
            cnt_f = _lane_sum((m0 + m1) + (m2 + m3), lanes)
            nzero = jnp.float32(L) - cnt_f
            length = jnp.maximum(cnt_f, jnp.float32(1e-9))
            avg0 = (acc0 - nzero * t00) / length
            avg1 = (acc1 - nzero * t01) / length

            sv = _lane_sum(avg0 * avg0 + avg1 * avg1, lanes)
            sv = jnp.maximum(sv, jnp.float32(1e-24))
            y = lax.bitcast_convert_type(
                jnp.int32(0x5F3759DF)
                - (lax.bitcast_convert_type(sv, jnp.int32) >> 1),
                jnp.float32)
            half = jnp.float32(0.5) * sv
            for _ in range(3):
                y = y * (jnp.float32(1.5) - half * y * y)
            y = y * jnp.minimum(cnt_f, jnp.float32(1.0))

            outc_v[r, pl.ds(0, 16)] = avg0 * y
            outc_v[r, pl.ds(16, 16)] = avg1 * y
            return rcarry

        lax.fori_loop(0, CHUNK_ROWS, row_body, 0)

        pltpu.sync_copy(outc_v, out.at[pl.ds(crow, CHUNK_ROWS)])
        return carry

    lax.fori_loop(0, CHUNKS, chunk_body, 0)


@jax.jit
def _tower(xflat, table):
    mesh = plsc.VectorSubcoreMesh(core_axis_name="c", subcore_axis_name="s")
    return pl.kernel(
        _tower_kernel,
        mesh=mesh,
        out_type=jax.ShapeDtypeStruct((B, DIM), jnp.float32),
        scratch_types=[
            pltpu.VMEM((GATHERS, GATHER_BATCH), jnp.int32),
            pltpu.VMEM((IDX_PER_CHUNK,), jnp.int32),
            pltpu.VMEM((IDX_PER_CHUNK, GDIM), jnp.float32),
            pltpu.VMEM((CHUNK_ROWS, DIM), jnp.float32),
            pltpu.VMEM((GDIM,), jnp.float32),
            pltpu.SemaphoreType.DMA,
        ],
    )(xflat, table)


def kernel(x, table):
    x = x.astype(jnp.int32)
    xflat = x.reshape(B * L)
    table_pad = jnp.pad(table, ((0, 0), (0, GDIM - DIM)))
    return _tower(xflat, table_pad)


# speedup vs baseline: 1.2500x; 1.2500x over previous
import functools

import jax
import jax.numpy as jnp
from jax import lax
from jax.experimental import pallas as pl
from jax.experimental.pallas import tpu as pltpu
from jax.experimental.pallas import tpu_sc as plsc

VOCAB = 1000000
DIM = 32
B = 16384
L = 50

NUM_CORES = 2
NUM_SUBCORES = 16
NUM_WORKERS = NUM_CORES * NUM_SUBCORES

ROWS_PER_WORKER = B // NUM_WORKERS
CHUNK_ROWS = 32
CHUNKS = ROWS_PER_WORKER // CHUNK_ROWS
PAIRS = CHUNKS // 2
IDX_PER_CHUNK = CHUNK_ROWS * L
GATHER_BATCH = 80
GATHERS = IDX_PER_CHUNK // GATHER_BATCH

_GATHER_DNUMS = lax.GatherDimensionNumbers(
    offset_dims=(), collapsed_slice_dims=(0,), start_index_map=(0,))


def _perm16(v, perm):
    return lax.gather(v, perm[:, None], _GATHER_DNUMS, (1,),
                      mode=lax.GatherScatterMode.PROMISE_IN_BOUNDS)


def _lane_sum(v, lanes):
    for k in (1, 2, 4, 8):
        v = v + _perm16(v, lanes ^ k)
    return v


def _tower_kernel(xflat, table, out, idxg0, idxg1, idxf0, idxf1, rows0,
                  rows1, outc0, outc1, t0_v, semi, semg0, semg1):
    wid = lax.axis_index("s") * NUM_CORES + lax.axis_index("c")
    base_row = wid * ROWS_PER_WORKER

    pltpu.sync_copy(table.at[0], t0_v)

    lanes = lax.iota(jnp.int32, 16)
    one = jnp.full((16,), 1.0, jnp.float32)
    zrow = jnp.full((16,), 0.0, jnp.float32)
    lt2 = jnp.where(lanes < 2, one, zrow)

    def stage_and_fire(c, idxg, idxf, rows, semg):
        foff = (base_row + c * CHUNK_ROWS) * L
        icps = []
        for j in range(GATHERS):
            icps.append(pltpu.async_copy(
                xflat.at[pl.ds(pl.multiple_of(foff + j * GATHER_BATCH, 8),
                               GATHER_BATCH)],
                idxg.at[j], semi))
        for cp in icps:
            cp.wait()
        pltpu.sync_copy(xflat.at[pl.ds(pl.multiple_of(foff, 8),
                                       IDX_PER_CHUNK)], idxf)
        for j in range(GATHERS):
            pltpu.async_copy(
                table.at[idxg.at[j]],
                rows.at[pl.ds(j * GATHER_BATCH, GATHER_BATCH)],
                semg)

    def wait_gathers(rows, semg):
        pltpu.make_async_copy(table.at[pl.ds(0, IDX_PER_CHUNK)], rows,
                              semg).wait()

    def compute(c, idxf, rows, outc):
        t00 = t0_v[pl.ds(0, 16)]
        t01 = t0_v[pl.ds(16, 16)]

        def row_body(r, rcarry):
            fo = r * L
            zero = jnp.zeros((16,), jnp.float32)
            a0 = [zero, zero, zero, zero]
            a1 = [zero, zero, zero, zero]
            for l in range(L):
                a0[l & 3] = a0[l & 3] + rows[fo + l, pl.ds(0, 16)]
                a1[l & 3] = a1[l & 3] + rows[fo + l, pl.ds(16, 16)]
            acc0 = (a0[0] + a0[1]) + (a0[2] + a0[3])
            acc1 = (a1[0] + a1[1]) + (a1[2] + a1[3])

            i0 = idxf[pl.ds(fo, 16)]
            i1 = idxf[pl.ds(fo + 16, 16)]
            i2 = idxf[pl.ds(fo + 32, 16)]
            i3 = idxf[pl.ds(fo + 34, 16)]
            m0 = jnp.where(i0 != 0, one, zrow)
            m1 = jnp.where(i1 != 0, one, zrow)
            m2 = jnp.where(i2 != 0, lt2, zrow)
            m3 = jnp.where(i3 != 0, one, zrow)
            cnt_f = _lane_sum((m0 + m1) + (m2 + m3), lanes)
            nzero = jnp.float32(L) - cnt_f
            length = jnp.maximum(cnt_f, jnp.float32(1e-9))
            avg0 = (acc0 - nzero * t00) / length
            avg1 = (acc1 - nzero * t01) / length

            sv = _lane_sum(avg0 * avg0 + avg1 * avg1, lanes)
            sv = jnp.maximum(sv, jnp.float32(1e-24))
            y = lax.bitcast_convert_type(
                jnp.int32(0x5F3759DF)
                - (lax.bitcast_convert_type(sv, jnp.int32) >> 1),
                jnp.float32)
            half = jnp.float32(0.5) * sv
            for _ in range(3):
                y = y * (jnp.float32(1.5) - half * y * y)
            y = y * jnp.minimum(cnt_f, jnp.float32(1.0))

            outc[r, pl.ds(0, 16)] = avg0 * y
            outc[r, pl.ds(16, 16)] = avg1 * y
            return rcarry

        lax.fori_loop(0, CHUNK_ROWS, row_body, 0)
        pltpu.sync_copy(outc, out.at[pl.ds(base_row + c * CHUNK_ROWS,
                                           CHUNK_ROWS)])

    stage_and_fire(0, idxg0, idxf0, rows0, semg0)
    stage_and_fire(1, idxg1, idxf1, rows1, semg1)

    def pair_body(p, carry):
        c0 = p * 2
        wait_gathers(rows0, semg0)
        compute(c0, idxf0, rows0, outc0)

        @pl.when(p < PAIRS - 1)
        def _():
            stage_and_fire(c0 + 2, idxg0, idxf0, rows0, semg0)

        wait_gathers(rows1, semg1)
        compute(c0 + 1, idxf1, rows1, outc1)

        @pl.when(p < PAIRS - 1)
        def _():
            stage_and_fire(c0 + 3, idxg1, idxf1, rows1, semg1)

        return carry

    lax.fori_loop(0, PAIRS, pair_body, 0)


@jax.jit
def _tower(xflat, table):
    mesh = plsc.VectorSubcoreMesh(core_axis_name="c", subcore_axis_name="s")
    return pl.kernel(
        _tower_kernel,
        mesh=mesh,
        compiler_params=pltpu.CompilerParams(use_tc_tiling_on_sc=False),
        out_type=jax.ShapeDtypeStruct((B, DIM), jnp.float32),
        scratch_types=[
            pltpu.VMEM((GATHERS, GATHER_BATCH), jnp.int32),
            pltpu.VMEM((GATHERS, GATHER_BATCH), jnp.int32),
            pltpu.VMEM((IDX_PER_CHUNK,), jnp.int32),
            pltpu.VMEM((IDX_PER_CHUNK,), jnp.int32),
            pltpu.VMEM((IDX_PER_CHUNK, DIM), jnp.float32),
            pltpu.VMEM((IDX_PER_CHUNK, DIM), jnp.float32),
            pltpu.VMEM((CHUNK_ROWS, DIM), jnp.float32),
            pltpu.VMEM((CHUNK_ROWS, DIM), jnp.float32),
            pltpu.VMEM((DIM,), jnp.float32),
            pltpu.SemaphoreType.DMA,
            pltpu.SemaphoreType.DMA,
            pltpu.SemaphoreType.DMA,
        ],
    )(xflat, table)


def kernel(x, table):
    x = x.astype(jnp.int32)
    xflat = x.reshape(B * L)
    return _tower(xflat, table)


# speedup vs baseline: 1.2971x; 1.0376x over previous
import functools

import jax
import jax.numpy as jnp
from jax import lax
from jax.experimental import pallas as pl
from jax.experimental.pallas import tpu as pltpu
from jax.experimental.pallas import tpu_sc as plsc

VOCAB = 1000000
DIM = 32
B = 16384
L = 50

NUM_CORES = 2
NUM_SUBCORES = 16
NUM_WORKERS = NUM_CORES * NUM_SUBCORES

ROWS_PER_WORKER = B // NUM_WORKERS
CHUNK_ROWS = 32
CHUNKS = ROWS_PER_WORKER // CHUNK_ROWS
PAIRS = CHUNKS // 2
IDX_PER_CHUNK = CHUNK_ROWS * L
GATHER_BATCH = 80
GATHERS = IDX_PER_CHUNK // GATHER_BATCH

_GATHER_DNUMS = lax.GatherDimensionNumbers(
    offset_dims=(), collapsed_slice_dims=(0,), start_index_map=(0,))


def _perm16(v, perm):
    return lax.gather(v, perm[:, None], _GATHER_DNUMS, (1,),
                      mode=lax.GatherScatterMode.PROMISE_IN_BOUNDS)


def _lane_sum(v, lanes):
    for k in (1, 2, 4, 8):
        v = v + _perm16(v, lanes ^ k)
    return v


def _tower_kernel(xflat, table, out, idxg0, idxg1, idxf0, idxf1, rows0,
                  rows1, outc0, outc1, t0_v, semg0, semg1):
    wid = lax.axis_index("s") * NUM_CORES + lax.axis_index("c")
    base_row = wid * ROWS_PER_WORKER

    pltpu.sync_copy(table.at[0], t0_v)

    lanes = lax.iota(jnp.int32, 16)
    one = jnp.full((16,), 1.0, jnp.float32)
    zrow = jnp.full((16,), 0.0, jnp.float32)
    lt2 = jnp.where(lanes < 2, one, zrow)

    def stage_and_fire(c, idxg, idxf, rows, semg):
        foff = (base_row + c * CHUNK_ROWS) * L
        pltpu.sync_copy(xflat.at[pl.ds(pl.multiple_of(foff, 8),
                                       IDX_PER_CHUNK)], idxf)
        for j in range(GATHERS):
            for s in range(GATHER_BATCH // 16):
                g = idxf[pl.ds(j * GATHER_BATCH + s * 16, 16)]
                idxg[j, pl.ds(s * 16, 16)] = g * 4
        for j in range(GATHERS):
            pltpu.async_copy(
                table.at[idxg.at[j]],
                rows.at[pl.ds(j * GATHER_BATCH, GATHER_BATCH)],
                semg)

    def wait_gathers(rows, semg):
        pltpu.make_async_copy(table.at[pl.ds(0, IDX_PER_CHUNK)], rows,
                              semg).wait()

    def compute(c, idxf, rows, outc):
        t00 = t0_v[pl.ds(0, 16)]
        t01 = t0_v[pl.ds(16, 16)]

        def row_body(r, rcarry):
            fo = r * L
            zero = jnp.zeros((16,), jnp.float32)
            a0 = [zero, zero, zero, zero]
            a1 = [zero, zero, zero, zero]
            for l in range(L):
                a0[l & 3] = a0[l & 3] + rows[fo + l, pl.ds(0, 16)]
                a1[l & 3] = a1[l & 3] + rows[fo + l, pl.ds(16, 16)]
            acc0 = (a0[0] + a0[1]) + (a0[2] + a0[3])
            acc1 = (a1[0] + a1[1]) + (a1[2] + a1[3])

            i0 = idxf[pl.ds(fo, 16)]
            i1 = idxf[pl.ds(fo + 16, 16)]
            i2 = idxf[pl.ds(fo + 32, 16)]
            i3 = idxf[pl.ds(fo + 34, 16)]
            m0 = jnp.where(i0 != 0, one, zrow)
            m1 = jnp.where(i1 != 0, one, zrow)
            m2 = jnp.where(i2 != 0, lt2, zrow)
            m3 = jnp.where(i3 != 0, one, zrow)
            cnt_f = _lane_sum((m0 + m1) + (m2 + m3), lanes)
            nzero = jnp.float32(L) - cnt_f
            length = jnp.maximum(cnt_f, jnp.float32(1e-9))
            avg0 = (acc0 - nzero * t00) / length
            avg1 = (acc1 - nzero * t01) / length

            sv = _lane_sum(avg0 * avg0 + avg1 * avg1, lanes)
            sv = jnp.maximum(sv, jnp.float32(1e-24))
            y = lax.bitcast_convert_type(
                jnp.int32(0x5F3759DF)
                - (lax.bitcast_convert_type(sv, jnp.int32) >> 1),
                jnp.float32)
            half = jnp.float32(0.5) * sv
            for _ in range(3):
                y = y * (jnp.float32(1.5) - half * y * y)
            y = y * jnp.minimum(cnt_f, jnp.float32(1.0))

            outc[r, pl.ds(0, 16)] = avg0 * y
            outc[r, pl.ds(16, 16)] = avg1 * y
            return rcarry

        lax.fori_loop(0, CHUNK_ROWS, row_body, 0)
        pltpu.sync_copy(outc, out.at[pl.ds(base_row + c * CHUNK_ROWS,
                                           CHUNK_ROWS)])

    stage_and_fire(0, idxg0, idxf0, rows0, semg0)
    stage_and_fire(1, idxg1, idxf1, rows1, semg1)

    def pair_body(p, carry):
        c0 = p * 2
        wait_gathers(rows0, semg0)
        compute(c0, idxf0, rows0, outc0)

        @pl.when(p < PAIRS - 1)
        def _():
            stage_and_fire(c0 + 2, idxg0, idxf0, rows0, semg0)

        wait_gathers(rows1, semg1)
        compute(c0 + 1, idxf1, rows1, outc1)

        @pl.when(p < PAIRS - 1)
        def _():
            stage_and_fire(c0 + 3, idxg1, idxf1, rows1, semg1)

        return carry

    lax.fori_loop(0, PAIRS, pair_body, 0)


@jax.jit
def _tower(xflat, table):
    mesh = plsc.VectorSubcoreMesh(core_axis_name="c", subcore_axis_name="s")
    return pl.kernel(
        _tower_kernel,
        mesh=mesh,
        compiler_params=pltpu.CompilerParams(use_tc_tiling_on_sc=False),
        out_type=jax.ShapeDtypeStruct((B, DIM), jnp.float32),
        scratch_types=[
            pltpu.VMEM((GATHERS, GATHER_BATCH), jnp.int32),
            pltpu.VMEM((GATHERS, GATHER_BATCH), jnp.int32),
            pltpu.VMEM((IDX_PER_CHUNK,), jnp.int32),
            pltpu.VMEM((IDX_PER_CHUNK,), jnp.int32),
            pltpu.VMEM((IDX_PER_CHUNK, DIM), jnp.float32),
            pltpu.VMEM((IDX_PER_CHUNK, DIM), jnp.float32),
            pltpu.VMEM((CHUNK_ROWS, DIM), jnp.float32),
            pltpu.VMEM((CHUNK_ROWS, DIM), jnp.float32),
            pltpu.VMEM((DIM,), jnp.float32),
            pltpu.SemaphoreType.DMA,
            pltpu.SemaphoreType.DMA,
        ],
    )(xflat, table)


def kernel(x, table):
    x = x.astype(jnp.int32)
    xflat = x.reshape(B * L)
    pad128 = jnp.zeros((VOCAB, 4 * DIM), jnp.float32)
    pad128 = lax.dynamic_update_slice(pad128, table, (0, 0))
    table4 = pad128.reshape(4 * VOCAB, DIM)
    return _tower(xflat, table4)


# speedup vs baseline: 1.3085x; 1.0088x over previous
import functools

import jax
import jax.numpy as jnp
from jax import lax
from jax.experimental import pallas as pl
from jax.experimental.pallas import tpu as pltpu
from jax.experimental.pallas import tpu_sc as plsc

VOCAB = 1000000
DIM = 32
B = 16384
L = 50

NUM_CORES = 2
NUM_SUBCORES = 16
NUM_WORKERS = NUM_CORES * NUM_SUBCORES

ROWS_PER_WORKER = B // NUM_WORKERS
IDX_PER_WORKER = ROWS_PER_WORKER * L
CHUNK_ROWS = 16
CHUNKS = ROWS_PER_WORKER // CHUNK_ROWS
PAIRS = CHUNKS // 2
IDX_PER_CHUNK = CHUNK_ROWS * L
GATHER_BATCH = 80
GATHERS = IDX_PER_CHUNK // GATHER_BATCH

_GATHER_DNUMS = lax.GatherDimensionNumbers(
    offset_dims=(), collapsed_slice_dims=(0,), start_index_map=(0,))


def _perm16(v, perm):
    return lax.gather(v, perm[:, None], _GATHER_DNUMS, (1,),
                      mode=lax.GatherScatterMode.PROMISE_IN_BOUNDS)


def _lane_sum(v, lanes):
    for k in (1, 2, 4, 8):
        v = v + _perm16(v, lanes ^ k)
    return v


def _tower_kernel(xflat, table, out, xall, idxg0, idxg1, rows0, rows1,
                  outc0, outc1, t0_v, semg0, semg1, semo0, semo1):
    wid = lax.axis_index("s") * NUM_CORES + lax.axis_index("c")
    base_row = wid * ROWS_PER_WORKER

    pltpu.sync_copy(table.at[0], t0_v)
    pltpu.sync_copy(xflat.at[pl.ds(pl.multiple_of(base_row * L, 8),
                                   IDX_PER_WORKER)], xall)

    lanes = lax.iota(jnp.int32, 16)
    one = jnp.full((16,), 1.0, jnp.float32)
    zrow = jnp.full((16,), 0.0, jnp.float32)
    lt2 = jnp.where(lanes < 2, one, zrow)

    def fire(c, idxg, rows, semg):
        co = c * IDX_PER_CHUNK
        for j in range(GATHERS):
            for s in range(GATHER_BATCH // 16):
                g = xall[pl.ds(co + j * GATHER_BATCH + s * 16, 16)]
                idxg[j, pl.ds(s * 16, 16)] = g * 4
        for j in range(GATHERS):
            pltpu.async_copy(
                table.at[idxg.at[j]],
                rows.at[pl.ds(j * GATHER_BATCH, GATHER_BATCH)],
                semg)

    def wait_gathers(rows, semg):
        pltpu.make_async_copy(table.at[pl.ds(0, IDX_PER_CHUNK)], rows,
                              semg).wait()

    def drain_out(outc, semo):
        pltpu.make_async_copy(out.at[pl.ds(0, CHUNK_ROWS)], outc,
                              semo).wait()

    def compute(c, rows, outc, semo):
        t00 = t0_v[pl.ds(0, 16)]
        t01 = t0_v[pl.ds(16, 16)]
        co = c * IDX_PER_CHUNK

        @pl.when(c >= 2)
        def _():
            drain_out(outc, semo)

        def row_body(r, rcarry):
            fo = r * L
            zero = jnp.zeros((16,), jnp.float32)
            a0 = [zero, zero, zero, zero]
            a1 = [zero, zero, zero, zero]
            for l in range(L):
                a0[l & 3] = a0[l & 3] + rows[fo + l, pl.ds(0, 16)]
                a1[l & 3] = a1[l & 3] + rows[fo + l, pl.ds(16, 16)]
            acc0 = (a0[0] + a0[1]) + (a0[2] + a0[3])
            acc1 = (a1[0] + a1[1]) + (a1[2] + a1[3])

            i0 = xall[pl.ds(co + fo, 16)]
            i1 = xall[pl.ds(co + fo + 16, 16)]
            i2 = xall[pl.ds(co + fo + 32, 16)]
            i3 = xall[pl.ds(co + fo + 34, 16)]
            m0 = jnp.where(i0 != 0, one, zrow)
            m1 = jnp.where(i1 != 0, one, zrow)
            m2 = jnp.where(i2 != 0, lt2, zrow)
            m3 = jnp.where(i3 != 0, one, zrow)
            cnt_f = _lane_sum((m0 + m1) + (m2 + m3), lanes)
            nzero = jnp.float32(L) - cnt_f
            length = jnp.maximum(cnt_f, jnp.float32(1e-9))
            avg0 = (acc0 - nzero * t00) / length
            avg1 = (acc1 - nzero * t01) / length

            sv = _lane_sum(avg0 * avg0 + avg1 * avg1, lanes)
            sv = jnp.maximum(sv, jnp.float32(1e-24))
            y = lax.bitcast_convert_type(
                jnp.int32(0x5F3759DF)
                - (lax.bitcast_convert_type(sv, jnp.int32) >> 1),
                jnp.float32)
            half = jnp.float32(0.5) * sv
            for _ in range(3):
                y = y * (jnp.float32(1.5) - half * y * y)
            y = y * jnp.minimum(cnt_f, jnp.float32(1.0))

            outc[r, pl.ds(0, 16)] = avg0 * y
            outc[r, pl.ds(16, 16)] = avg1 * y
            return rcarry

        lax.fori_loop(0, CHUNK_ROWS, row_body, 0)
        pltpu.async_copy(outc, out.at[pl.ds(base_row + c * CHUNK_ROWS,
                                            CHUNK_ROWS)], semo)

    fire(0, idxg0, rows0, semg0)
    fire(1, idxg1, rows1, semg1)

    def pair_body(p, carry):
        c0 = p * 2
        wait_gathers(rows0, semg0)
        compute(c0, rows0, outc0, semo0)

        @pl.when(p < PAIRS - 1)
        def _():
            fire(c0 + 2, idxg0, rows0, semg0)

        wait_gathers(rows1, semg1)
        compute(c0 + 1, rows1, outc1, semo1)

        @pl.when(p < PAIRS - 1)
        def _():
            fire(c0 + 3, idxg1, rows1, semg1)

        return carry

    lax.fori_loop(0, PAIRS, pair_body, 0)

    drain_out(outc0, semo0)
    drain_out(outc1, semo1)


@jax.jit
def _tower(xflat, table):
    mesh = plsc.VectorSubcoreMesh(core_axis_name="c", subcore_axis_name="s")
    return pl.kernel(
        _tower_kernel,
        mesh=mesh,
        compiler_params=pltpu.CompilerParams(use_tc_tiling_on_sc=False),
        out_type=jax.ShapeDtypeStruct((B, DIM), jnp.float32),
        scratch_types=[
            pltpu.VMEM((IDX_PER_WORKER,), jnp.int32),
            pltpu.VMEM((GATHERS, GATHER_BATCH), jnp.int32),
            pltpu.VMEM((GATHERS, GATHER_BATCH), jnp.int32),
            pltpu.VMEM((IDX_PER_CHUNK, DIM), jnp.float32),
            pltpu.VMEM((IDX_PER_CHUNK, DIM), jnp.float32),
            pltpu.VMEM((CHUNK_ROWS, DIM), jnp.float32),
            pltpu.VMEM((CHUNK_ROWS, DIM), jnp.float32),
            pltpu.VMEM((DIM,), jnp.float32),
            pltpu.SemaphoreType.DMA,
            pltpu.SemaphoreType.DMA,
            pltpu.SemaphoreType.DMA,
            pltpu.SemaphoreType.DMA,
        ],
    )(xflat, table)


def kernel(x, table):
    x = x.astype(jnp.int32)
    xflat = x.reshape(B * L)
    pad128 = jnp.concatenate(
        [table, jnp.zeros((VOCAB, 3 * DIM), jnp.float32)], axis=1)
    table4 = pad128.reshape(4 * VOCAB, DIM)
    return _tower(xflat, table4)


# speedup vs baseline: 1.9098x; 1.4596x over previous
import functools

import jax
import jax.numpy as jnp
from jax import lax
from jax.experimental import pallas as pl
from jax.experimental.pallas import tpu as pltpu
from jax.experimental.pallas import tpu_sc as plsc
from jax.experimental.layout import Format, Layout, with_layout_constraint

VOCAB = 1000000
DIM = 32
B = 16384
L = 50

NUM_CORES = 2
NUM_SUBCORES = 16
NUM_WORKERS = NUM_CORES * NUM_SUBCORES

ROWS_PER_WORKER = B // NUM_WORKERS
IDX_PER_WORKER = ROWS_PER_WORKER * L
CHUNK_ROWS = 16
CHUNKS = ROWS_PER_WORKER // CHUNK_ROWS
PAIRS = CHUNKS // 2
IDX_PER_CHUNK = CHUNK_ROWS * L
GATHER_BATCH = 80
GATHERS = IDX_PER_CHUNK // GATHER_BATCH

_GATHER_DNUMS = lax.GatherDimensionNumbers(
    offset_dims=(), collapsed_slice_dims=(0,), start_index_map=(0,))


def _perm16(v, perm):
    return lax.gather(v, perm[:, None], _GATHER_DNUMS, (1,),
                      mode=lax.GatherScatterMode.PROMISE_IN_BOUNDS)


def _lane_sum(v, lanes):
    for k in (1, 2, 4, 8):
        v = v + _perm16(v, lanes ^ k)
    return v


def _tower_kernel(xflat, table, out, xall, idxg0, idxg1, rows0, rows1,
                  outc0, outc1, t0_v, semg0, semg1, semo0, semo1):
    wid = lax.axis_index("s") * NUM_CORES + lax.axis_index("c")
    base_row = wid * ROWS_PER_WORKER

    pltpu.sync_copy(table.at[0], t0_v)
    pltpu.sync_copy(xflat.at[pl.ds(pl.multiple_of(base_row * L, 8),
                                   IDX_PER_WORKER)], xall)

    lanes = lax.iota(jnp.int32, 16)
    one = jnp.full((16,), 1.0, jnp.float32)
    zrow = jnp.full((16,), 0.0, jnp.float32)
    lt2 = jnp.where(lanes < 2, one, zrow)

    def fire(c, idxg, rows, semg):
        co = c * IDX_PER_CHUNK
        for j in range(GATHERS):
            for s in range(GATHER_BATCH // 16):
                g = xall[pl.ds(co + j * GATHER_BATCH + s * 16, 16)]
                idxg[j, pl.ds(s * 16, 16)] = g
        for j in range(GATHERS):
            pltpu.async_copy(
                table.at[idxg.at[j]],
                rows.at[pl.ds(j * GATHER_BATCH, GATHER_BATCH)],
                semg)

    def wait_gathers(rows, semg):
        pltpu.make_async_copy(table.at[pl.ds(0, IDX_PER_CHUNK)], rows,
                              semg).wait()

    def drain_out(outc, semo):
        pltpu.make_async_copy(out.at[pl.ds(0, CHUNK_ROWS)], outc,
                              semo).wait()

    def compute(c, rows, outc, semo):
        t00 = t0_v[pl.ds(0, 16)]
        t01 = t0_v[pl.ds(16, 16)]
        co = c * IDX_PER_CHUNK

        @pl.when(c >= 2)
        def _():
            drain_out(outc, semo)

        def row_body(r, rcarry):
            fo = r * L
            zero = jnp.zeros((16,), jnp.float32)
            a0 = [zero, zero, zero, zero]
            a1 = [zero, zero, zero, zero]
            for l in range(L):
                a0[l & 3] = a0[l & 3] + rows[fo + l, pl.ds(0, 16)]
                a1[l & 3] = a1[l & 3] + rows[fo + l, pl.ds(16, 16)]
            acc0 = (a0[0] + a0[1]) + (a0[2] + a0[3])
            acc1 = (a1[0] + a1[1]) + (a1[2] + a1[3])

            i0 = xall[pl.ds(co + fo, 16)]
            i1 = xall[pl.ds(co + fo + 16, 16)]
            i2 = xall[pl.ds(co + fo + 32, 16)]
            i3 = xall[pl.ds(co + fo + 34, 16)]
            m0 = jnp.where(i0 != 0, one, zrow)
            m1 = jnp.where(i1 != 0, one, zrow)
            m2 = jnp.where(i2 != 0, lt2, zrow)
            m3 = jnp.where(i3 != 0, one, zrow)
            cnt_f = _lane_sum((m0 + m1) + (m2 + m3), lanes)
            nzero = jnp.float32(L) - cnt_f
            length = jnp.maximum(cnt_f, jnp.float32(1e-9))
            avg0 = (acc0 - nzero * t00) / length
            avg1 = (acc1 - nzero * t01) / length

            sv = _lane_sum(avg0 * avg0 + avg1 * avg1, lanes)
            sv = jnp.maximum(sv, jnp.float32(1e-24))
            y = lax.bitcast_convert_type(
                jnp.int32(0x5F3759DF)
                - (lax.bitcast_convert_type(sv, jnp.int32) >> 1),
                jnp.float32)
            half = jnp.float32(0.5) * sv
            for _ in range(3):
                y = y * (jnp.float32(1.5) - half * y * y)
            y = y * jnp.minimum(cnt_f, jnp.float32(1.0))

            outc[r, pl.ds(0, 16)] = avg0 * y
            outc[r, pl.ds(16, 16)] = avg1 * y
            return rcarry

        lax.fori_loop(0, CHUNK_ROWS, row_body, 0)
        pltpu.async_copy(outc, out.at[pl.ds(base_row + c * CHUNK_ROWS,
                                            CHUNK_ROWS)], semo)

    fire(0, idxg0, rows0, semg0)
    fire(1, idxg1, rows1, semg1)

    def pair_body(p, carry):
        c0 = p * 2
        wait_gathers(rows0, semg0)
        compute(c0, rows0, outc0, semo0)

        @pl.when(p < PAIRS - 1)
        def _():
            fire(c0 + 2, idxg0, rows0, semg0)

        wait_gathers(rows1, semg1)
        compute(c0 + 1, rows1, outc1, semo1)

        @pl.when(p < PAIRS - 1)
        def _():
            fire(c0 + 3, idxg1, rows1, semg1)

        return carry

    lax.fori_loop(0, PAIRS, pair_body, 0)

    drain_out(outc0, semo0)
    drain_out(outc1, semo1)


@jax.jit
def _tower(xflat, table):
    mesh = plsc.VectorSubcoreMesh(core_axis_name="c", subcore_axis_name="s")
    return pl.kernel(
        _tower_kernel,
        mesh=mesh,
        compiler_params=pltpu.CompilerParams(use_tc_tiling_on_sc=False),
        out_type=jax.ShapeDtypeStruct((B, DIM), jnp.float32),
        scratch_types=[
            pltpu.VMEM((IDX_PER_WORKER,), jnp.int32),
            pltpu.VMEM((GATHERS, GATHER_BATCH), jnp.int32),
            pltpu.VMEM((GATHERS, GATHER_BATCH), jnp.int32),
            pltpu.VMEM((IDX_PER_CHUNK, DIM), jnp.float32),
            pltpu.VMEM((IDX_PER_CHUNK, DIM), jnp.float32),
            pltpu.VMEM((CHUNK_ROWS, DIM), jnp.float32),
            pltpu.VMEM((CHUNK_ROWS, DIM), jnp.float32),
            pltpu.VMEM((DIM,), jnp.float32),
            pltpu.SemaphoreType.DMA,
            pltpu.SemaphoreType.DMA,
            pltpu.SemaphoreType.DMA,
            pltpu.SemaphoreType.DMA,
        ],
    )(xflat, table)


def kernel(x, table):
    x = x.astype(jnp.int32)
    xflat = x.reshape(B * L)
    table_rm = with_layout_constraint(
        table, Layout(major_to_minor=(0, 1), tiling=((8,),)))
    return _tower(xflat, table_rm)
